# baseline (speedup 1.0000x reference)
import functools

import jax
import jax.numpy as jnp
from jax import lax
from jax.experimental import pallas as pl
from jax.experimental.pallas import tpu as pltpu
from jax.experimental.pallas import tpu_sc as plsc

EMBEDDING_DIM = 32

_info = plsc.get_sparse_core_info()
_NC, _NS = _info.num_cores, _info.num_subcores
_NW = _NC * _NS


@functools.lru_cache(maxsize=None)
def _build_idx(batch, fields):
    assert batch % _NW == 0
    b_per_w = batch // _NW
    mesh = plsc.VectorSubcoreMesh(core_axis_name="c", subcore_axis_name="s")

    @functools.partial(
        pl.kernel,
        mesh=mesh,
        out_type=jax.ShapeDtypeStruct((fields, batch), jnp.int32),
        compiler_params=pltpu.CompilerParams(use_tc_tiling_on_sc=True),
        scratch_types=[pltpu.SemaphoreType.DMA],
    )
    def k(ids_hbm, out_hbm, sem):
        wid = lax.axis_index("s") * _NC + lax.axis_index("c")
        b0 = pl.multiple_of(wid * b_per_w, b_per_w)
        hs = []
        for f in range(fields):
            hs.append(pltpu.async_copy(
                ids_hbm.at[pl.ds(b0, b_per_w), f],
                out_hbm.at[f, pl.ds(b0, b_per_w)], sem))
        for h in hs:
            h.wait()

    return k


@functools.lru_cache(maxsize=None)
def _build_gather(fields, batch, dim, nbuf, inflight):
    assert batch % _NW == 0
    b_per_w = batch // _NW
    chunk = b_per_w
    n_chunks = fields
    assert inflight < nbuf
    mesh = plsc.VectorSubcoreMesh(core_axis_name="c", subcore_axis_name="s")

    @functools.partial(
        pl.kernel,
        mesh=mesh,
        out_type=jax.ShapeDtypeStruct((batch, fields, dim), jnp.float32),
        compiler_params=pltpu.CompilerParams(use_tc_tiling_on_sc=False),
        scratch_types=(
            [pltpu.VMEM((fields * b_per_w,), jnp.int32)]
            + [pltpu.VMEM((chunk, dim), jnp.float32) for _ in range(nbuf)]
            + [pltpu.SemaphoreType.DMA for _ in range(2 * nbuf + 1)]
        ),
    )
    def k(table_hbm, idx_hbm, out_hbm, idx_v, *rest):
        bufs = rest[:nbuf]
        gsems = rest[nbuf:2 * nbuf]
        osems = rest[2 * nbuf:3 * nbuf]
        isem = rest[3 * nbuf]
        wid = lax.axis_index("s") * _NC + lax.axis_index("c")
        b0 = pl.multiple_of(wid * b_per_w, b_per_w)

        idx_h = [None] * fields
        for f in range(fields):
            idx_h[f] = pltpu.async_copy(
                idx_hbm.at[f, pl.ds(b0, b_per_w)],
                idx_v.at[pl.ds(f * b_per_w, b_per_w)], isem)

        gather_h = [None] * n_chunks
        out_h = [None] * n_chunks

        def start_gather(c):
            s = c % nbuf
            gather_h[c] = pltpu.async_copy(
                table_hbm.at[idx_v.at[pl.ds(c * chunk, chunk)]],
                bufs[s], gsems[s])

        for f in range(fields):
            idx_h[f].wait()
        for j in range(min(inflight, n_chunks)):
            start_gather(j)
        for c in range(n_chunks):
            f = c + inflight
            if f < n_chunks:
                prev = f - nbuf
                if prev >= 0:
                    out_h[prev].wait()
                start_gather(f)
            gather_h[c].wait()
            s = c % nbuf
            out_h[c] = pltpu.async_copy(
                bufs[s], out_hbm.at[pl.ds(b0, chunk), c, :],
                osems[s])
        for c in range(max(0, n_chunks - nbuf), n_chunks):
            out_h[c].wait()

    return k


def kernel(token_ids, weight):
    batch, fields = token_ids.shape
    n_rows, dim = weight.shape
    idx_t = token_ids.T.astype(jnp.int32)
    out = _build_gather(fields, batch, dim, 6, 4)(weight, idx_t)
    return out


# speedup vs baseline: 1.0580x; 1.0580x over previous
import functools

import jax
import jax.numpy as jnp
from jax import lax
from jax.experimental import pallas as pl
from jax.experimental.pallas import tpu as pltpu
from jax.experimental.pallas import tpu_sc as plsc

EMBEDDING_DIM = 32

_info = plsc.get_sparse_core_info()
_NC, _NS = _info.num_cores, _info.num_subcores
_NW = _NC * _NS


@functools.lru_cache(maxsize=None)
def _build_gather(fields, batch, dim, nbuf, inflight):
    assert batch % _NW == 0
    b_per_w = batch // _NW
    chunk = b_per_w
    n_chunks = fields
    assert inflight < nbuf
    mesh = plsc.VectorSubcoreMesh(core_axis_name="c", subcore_axis_name="s")

    @functools.partial(
        pl.kernel,
        mesh=mesh,
        out_type=jax.ShapeDtypeStruct((fields * batch, dim), jnp.float32),
        compiler_params=pltpu.CompilerParams(use_tc_tiling_on_sc=False),
        scratch_types=(
            [pltpu.VMEM((fields * b_per_w,), jnp.int32)]
            + [pltpu.VMEM((chunk, dim), jnp.float32) for _ in range(nbuf)]
            + [pltpu.SemaphoreType.DMA for _ in range(2 * nbuf + 1)]
        ),
    )
    def k(table_hbm, idx_hbm, out_hbm, idx_v, *rest):
        bufs = rest[:nbuf]
        gsems = rest[nbuf:2 * nbuf]
        osems = rest[2 * nbuf:3 * nbuf]
        isem = rest[3 * nbuf]
        wid = lax.axis_index("s") * _NC + lax.axis_index("c")
        b0 = pl.multiple_of(wid * b_per_w, b_per_w)

        idx_h = [None] * fields
        for f in range(fields):
            idx_h[f] = pltpu.async_copy(
                idx_hbm.at[f, pl.ds(b0, b_per_w)],
                idx_v.at[pl.ds(f * b_per_w, b_per_w)], isem)

        gather_h = [None] * n_chunks
        out_h = [None] * n_chunks

        def start_gather(c):
            s = c % nbuf
            gather_h[c] = pltpu.async_copy(
                table_hbm.at[idx_v.at[pl.ds(c * chunk, chunk)]],
                bufs[s], gsems[s])

        for f in range(fields):
            idx_h[f].wait()
        for j in range(min(inflight, n_chunks)):
            start_gather(j)
        for c in range(n_chunks):
            f = c + inflight
            if f < n_chunks:
                prev = f - nbuf
                if prev >= 0:
                    out_h[prev].wait()
                start_gather(f)
            gather_h[c].wait()
            s = c % nbuf
            out_h[c] = pltpu.async_copy(
                bufs[s], out_hbm.at[pl.ds(c * batch + b0, chunk)],
                osems[s])
        for c in range(max(0, n_chunks - nbuf), n_chunks):
            out_h[c].wait()

    return k


def kernel(token_ids, weight):
    batch, fields = token_ids.shape
    n_rows, dim = weight.shape
    idx_t = token_ids.T.astype(jnp.int32)
    out = _build_gather(fields, batch, dim, 6, 4)(weight, idx_t)
    return out.reshape(fields, batch, dim).transpose(1, 0, 2)


# speedup vs baseline: 1.0593x; 1.0012x over previous
import functools

import jax
import jax.numpy as jnp
from jax import lax
from jax.experimental import pallas as pl
from jax.experimental.pallas import tpu as pltpu
from jax.experimental.pallas import tpu_sc as plsc

EMBEDDING_DIM = 32

_info = plsc.get_sparse_core_info()
_NC, _NS = _info.num_cores, _info.num_subcores
_NW = _NC * _NS


@functools.lru_cache(maxsize=None)
def _build_gather(fields, batch, dim, nbuf, inflight):
    assert batch % _NW == 0
    b_per_w = batch // _NW
    chunk = b_per_w
    n_chunks = fields
    assert inflight < nbuf
    mesh = plsc.VectorSubcoreMesh(core_axis_name="c", subcore_axis_name="s")

    @functools.partial(
        pl.kernel,
        mesh=mesh,
        out_type=jax.ShapeDtypeStruct((fields * batch, dim), jnp.float32),
        compiler_params=pltpu.CompilerParams(use_tc_tiling_on_sc=False),
        scratch_types=(
            [pltpu.VMEM((fields * b_per_w,), jnp.int32)]
            + [pltpu.VMEM((chunk, dim), jnp.float32) for _ in range(nbuf)]
            + [pltpu.SemaphoreType.DMA for _ in range(2 * nbuf + 1)]
        ),
    )
    def k(table_hbm, idx_hbm, out_hbm, idx_v, *rest):
        bufs = rest[:nbuf]
        gsems = rest[nbuf:2 * nbuf]
        osems = rest[2 * nbuf:3 * nbuf]
        isem = rest[3 * nbuf]
        wid = lax.axis_index("s") * _NC + lax.axis_index("c")
        b0 = pl.multiple_of(wid * b_per_w, b_per_w)

        idx_h = [None] * fields
        for f in range(fields):
            idx_h[f] = pltpu.async_copy(
                idx_hbm.at[f, pl.ds(b0, b_per_w)],
                idx_v.at[pl.ds(f * b_per_w, b_per_w)], isem)

        gather_h = [None] * n_chunks
        out_h = [None] * n_chunks

        def start_gather(c):
            s = c % nbuf
            gather_h[c] = pltpu.async_copy(
                table_hbm.at[idx_v.at[pl.ds(c * chunk, chunk)]],
                bufs[s], gsems[s])

        for f in range(fields):
            idx_h[f].wait()
        for j in range(min(inflight, n_chunks)):
            start_gather(j)
        for c in range(n_chunks):
            f = c + inflight
            if f < n_chunks:
                prev = f - nbuf
                if prev >= 0:
                    out_h[prev].wait()
                start_gather(f)
            gather_h[c].wait()
            s = c % nbuf
            out_h[c] = pltpu.async_copy(
                bufs[s], out_hbm.at[pl.ds(c * batch + b0, chunk)],
                osems[s])
        for c in range(max(0, n_chunks - nbuf), n_chunks):
            out_h[c].wait()

    return k


def kernel(token_ids, weight):
    batch, fields = token_ids.shape
    n_rows, dim = weight.shape
    idx_t = token_ids.T.astype(jnp.int32)
    out = _build_gather(fields, batch, dim, 7, 5)(weight, idx_t)
    return out.reshape(fields, batch, dim).transpose(1, 0, 2)
